# baseline (speedup 1.0000x reference)
import functools

import jax
import jax.numpy as jnp
from jax.experimental import pallas as pl
from jax.experimental.pallas import tpu as pltpu


def _final_body(x5_ref, w_ref, out_ref):
    x5 = x5_ref[...]
    o = jax.lax.dot_general(x5, w_ref[...], (((1,), (1,)), ((), ())),
                            preferred_element_type=jnp.float32)
    out_ref[...] = jax.nn.sigmoid(o)


def _final_stage(x5, lw, lb, l2w, l2b):
    wc = l2w @ lw
    bc = lb @ l2w.T + l2b
    w_aug = jnp.concatenate([wc, bc[None, :]], axis=1)
    nb = x5.shape[0]
    pad = (-nb) % 128
    x5a = jnp.concatenate([x5, jnp.ones((nb, 1), x5.dtype)], axis=1)
    x5p = jnp.pad(x5a, ((0, pad), (0, 0)))
    npad = nb + pad
    grid = npad // 128
    out = pl.pallas_call(
        _final_body,
        grid=(grid,),
        in_specs=[
            pl.BlockSpec((128, x5p.shape[1]), lambda i: (i, 0)),
            pl.BlockSpec(w_aug.shape, lambda i: (0, 0)),
        ],
        out_specs=pl.BlockSpec((128, 1), lambda i: (i, 0)),
        out_shape=jax.ShapeDtypeStruct((npad, 1), jnp.float32),
    )(x5p, w_aug)
    return out[:nb, 0]


def kernel(x, edge_weight, params, edge_index):
    p = params
    row, col = edge_index[0], edge_index[1]
    N = x.shape[0]
    nb = N // 30
    w = edge_weight[:, 0]

    mu = x.mean(axis=0)
    mx = x.max(axis=0)
    x0 = (x - mu) / (mx - mu)

    def layer(xin, gcn, gin):
        z = xin @ p[gcn + "_W2"].T
        y = xin @ p[gin + "_W1"].T
        g = jax.ops.segment_sum(z[row] * w[:, None], col, num_segments=N)
        g = g + p[gcn + "_b2"]
        a = jax.ops.segment_sum(y[row], col, num_segments=N)
        t = jax.nn.relu((1.0 + p[gin + "_eps"]) * y + a + p[gin + "_b1"])
        i_ = jax.nn.relu(t @ p[gin + "_W2"].T + p[gin + "_b2"])
        return jax.nn.leaky_relu(jnp.concatenate([g, i_], axis=1),
                                 negative_slope=0.2)

    x1 = layer(x0, "gcn1", "gin1")
    x2 = layer(x1, "gcn2", "gin2")
    x3 = layer(x2, "gcn3", "gin3")

    hv = (x3 @ p["gat_W"].T)[:, 0]
    a0 = p["gat_att"][0, 0, 0]
    l = jax.nn.leaky_relu(a0 * hv, negative_slope=0.2)
    m = jnp.max(l)
    ev = jnp.exp(l - m)
    rowdeg = jax.ops.segment_sum(jnp.ones_like(w), row, num_segments=N)
    z_norm = jnp.sum(rowdeg * ev)
    q = hv * ev
    attw = jax.ops.segment_sum(q[row], col, num_segments=N) / z_norm

    x3a = x3 * attw[:, None]
    x5 = x3a.reshape(nb, 30, x3.shape[1]).sum(axis=1) / 30.0
    return _final_stage(x5, p["lin_W"], p["lin_b"], p["lin2_W"], p["lin2_b"])


# speedup vs baseline: 5.5288x; 5.5288x over previous
import functools

import jax
import jax.numpy as jnp
from jax import lax
from jax.experimental import pallas as pl
from jax.experimental.pallas import tpu as pltpu
from jax.experimental.pallas import tpu_sc as plsc

N_NODES = 49980
NPAD = 50048
E_EDGES = 799680
NW = 32
PER_W = 26624
EPAD = NW * PER_W
B_WIN = 1024
BI = B_WIN // 128
NWIN = PER_W // B_WIN


DC = 16


def _layer_pass(p_chunks, weighted, row2, col2, wx, zeros):
    nch = len(p_chunks)
    mesh = plsc.VectorSubcoreMesh(core_axis_name="c", subcore_axis_name="s")

    def body(*refs):
        ps = refs[:nch]
        row_hbm, col_hbm, w_hbm, z_hbm, out_hbm = refs[nch:nch + 5]
        (ridx0, ridx1, cidx0, cidx1, wv0, wv1, rows0, rows1, acc,
         semg0, semg1, sems0, sems1) = refs[nch + 5:]
        cid = lax.axis_index("c")
        sid = lax.axis_index("s")
        base128 = (sid * 2 + cid) * (PER_W // 128)
        ridx = (ridx0, ridx1)
        cidx = (cidx0, cidx1)
        wv = (wv0, wv1)
        rows = (rows0, rows1)
        semg = (semg0, semg1)
        sems = (sems0, sems1)

        def drain_scatter(b):
            pltpu.make_async_copy(ps[0].at[pl.ds(0, B_WIN)],
                                  rows[b], sems[b]).wait()

        def prefetch(p_hbm, wgt, i, b):
            wrow = base128 + i * BI
            pltpu.sync_copy(row_hbm.at[pl.ds(wrow, BI)], ridx[b])
            for j in range(BI):
                pltpu.async_copy(p_hbm.at[ridx[b].at[j]],
                                 rows[b].at[pl.ds(j * 128, 128)], semg[b])
            if wgt:
                pltpu.sync_copy(w_hbm.at[pl.ds(wrow * 128, B_WIN)], wv[b])

        def compute(p_hbm, wgt, i, b):
            wrow = base128 + i * BI
            pltpu.make_async_copy(p_hbm.at[pl.ds(0, B_WIN)],
                                  rows[b], semg[b]).wait()
            if wgt:
                def mul8(g, c2):
                    for u in range(8):
                        r = g * 8 + u
                        rows[b][r, :] = rows[b][r, :] * wv[b][r, :]
                    return c2
                lax.fori_loop(0, B_WIN // 8, mul8, 0)
            pltpu.sync_copy(col_hbm.at[pl.ds(wrow, BI)], cidx[b])
            for j in range(BI):
                pltpu.sync_copy(rows[b].at[pl.ds(j * 128, 128)],
                                acc.at[cidx[b].at[j]], add=True)

        half = NWIN // 2
        for k in range(nch):
            p_hbm = ps[k]
            wgt = weighted[k]
            prefetch(p_hbm, wgt, 0, 0)
            plsc.subcore_barrier()

            @pl.when(sid == 0)
            def _zero():
                pltpu.sync_copy(z_hbm, acc)

            plsc.subcore_barrier()

            def pair(t, carry):
                prefetch(p_hbm, wgt, 2 * t + 1, 1)
                compute(p_hbm, wgt, 2 * t, 0)

                @pl.when(t + 1 < half)
                def _pf():
                    prefetch(p_hbm, wgt, 2 * t + 2, 0)

                compute(p_hbm, wgt, 2 * t + 1, 1)
                return carry

            lax.fori_loop(0, half, pair, 0)
            plsc.subcore_barrier()

            @pl.when(sid == 0)
            def _out():
                pltpu.sync_copy(acc, out_hbm.at[k].at[cid])

    k = pl.kernel(
        body,
        mesh=mesh,
        compiler_params=pltpu.CompilerParams(use_tc_tiling_on_sc=False),
        out_type=jax.ShapeDtypeStruct((nch, 2, NPAD, DC), jnp.float32),
        scratch_types=[
            pltpu.VMEM((BI, 128), jnp.int32),
            pltpu.VMEM((BI, 128), jnp.int32),
            pltpu.VMEM((BI, 128), jnp.int32),
            pltpu.VMEM((BI, 128), jnp.int32),
            pltpu.VMEM((B_WIN, DC), jnp.float32),
            pltpu.VMEM((B_WIN, DC), jnp.float32),
            pltpu.VMEM((B_WIN, DC), jnp.float32),
            pltpu.VMEM((B_WIN, DC), jnp.float32),
            pltpu.VMEM_SHARED((NPAD, DC), jnp.float32),
            pltpu.SemaphoreType.DMA,
            pltpu.SemaphoreType.DMA,
            pltpu.SemaphoreType.DMA,
            pltpu.SemaphoreType.DMA,
        ],
    )
    return k(*p_chunks, row2, col2, wx, zeros)


def _dot_t(a, b):
    return jax.lax.dot_general(a, b, (((1,), (1,)), ((), ())),
                               preferred_element_type=jnp.float32)


NB_TC = NPAD // 128


def _first_stage(x0p, wg, wy):
    dout = wg.shape[0]
    nch = dout // DC

    def body(x_ref, wg_ref, wy_ref, *outs):
        x = x_ref[...]
        zn = _dot_t(x, wg_ref[...])
        yn = _dot_t(x, wy_ref[...])
        for c in range(nch):
            outs[c][...] = zn[:, c * DC:(c + 1) * DC]
        outs[nch][...] = yn

    outs = pl.pallas_call(
        body,
        grid=(NB_TC,),
        in_specs=[
            pl.BlockSpec((128, x0p.shape[1]), lambda i: (i, 0)),
            pl.BlockSpec(wg.shape, lambda i: (0, 0)),
            pl.BlockSpec(wy.shape, lambda i: (0, 0)),
        ],
        out_specs=[pl.BlockSpec((128, DC), lambda i: (i, 0))] * (nch + 1),
        out_shape=[jax.ShapeDtypeStruct((NPAD, DC), jnp.float32)] * (nch + 1),
    )(x0p, wg, wy)
    return list(outs)


def _combine_transform(us, ysc, wi2p, b2i, bg2, wng, wny, n_valid):
    ngcn = len(us) - 1
    doutn = wng.shape[0]
    nchn = doutn // DC

    def body(*refs):
        u_refs = refs[:ngcn + 1]
        ysc_ref, wi2_ref, b2i_ref, bg2_ref, wg_ref, wy_ref = \
            refs[ngcn + 1:ngcn + 7]
        outs = refs[ngcn + 7:]
        pid = pl.program_id(0)
        g = jnp.concatenate([u[0] + u[1] for u in u_refs[:ngcn]], axis=1)
        g = g + bg2_ref[...]
        a16 = u_refs[ngcn][0] + u_refs[ngcn][1]
        t16 = jax.nn.relu(ysc_ref[...] + a16)
        i_ = jax.nn.relu(_dot_t(t16, wi2_ref[...]) + b2i_ref[...])
        xl = jnp.concatenate([g, i_], axis=1)
        xl = jnp.where(xl >= 0, xl, 0.2 * xl)
        rid = jax.lax.broadcasted_iota(jnp.int32, xl.shape, 0) + pid * 128
        xl = jnp.where(rid < n_valid, xl, 0.0)
        zn = _dot_t(xl, wg_ref[...])
        yn = _dot_t(xl, wy_ref[...])
        for c in range(nchn):
            outs[c][...] = zn[:, c * DC:(c + 1) * DC]
        outs[nchn][...] = yn

    outs = pl.pallas_call(
        body,
        grid=(NB_TC,),
        in_specs=(
            [pl.BlockSpec((2, 128, DC), lambda i: (0, i, 0))] * (ngcn + 1)
            + [pl.BlockSpec((128, DC), lambda i: (i, 0))]
            + [pl.BlockSpec(w.shape, lambda i: (0, 0))
               for w in (wi2p, b2i, bg2, wng, wny)]
        ),
        out_specs=[pl.BlockSpec((128, DC), lambda i: (i, 0))] * (nchn + 1),
        out_shape=[jax.ShapeDtypeStruct((NPAD, DC), jnp.float32)]
        * (nchn + 1),
    )(*us, ysc, wi2p, b2i, bg2, wng, wny)
    return list(outs)


def _combine3(us, ysc, wi2p, b2i, bg2, gcat, n_valid):
    ngcn = len(us) - 1

    def body(*refs):
        u_refs = refs[:ngcn + 1]
        ysc_ref, wi2_ref, b2i_ref, bg2_ref, g_ref = refs[ngcn + 1:ngcn + 6]
        x3_ref, hl_ref = refs[ngcn + 6:]
        pid = pl.program_id(0)
        g = jnp.concatenate([u[0] + u[1] for u in u_refs[:ngcn]], axis=1)
        g = g + bg2_ref[...]
        a16 = u_refs[ngcn][0] + u_refs[ngcn][1]
        t16 = jax.nn.relu(ysc_ref[...] + a16)
        i_ = jax.nn.relu(_dot_t(t16, wi2_ref[...]) + b2i_ref[...])
        xl = jnp.concatenate([g, i_], axis=1)
        xl = jnp.where(xl >= 0, xl, 0.2 * xl)
        rid = jax.lax.broadcasted_iota(jnp.int32, xl.shape, 0) + pid * 128
        xl = jnp.where(rid < n_valid, xl, 0.0)
        x3_ref[...] = xl
        hl_ref[...] = jax.lax.dot_general(
            xl, g_ref[...], (((1,), (0,)), ((), ())),
            preferred_element_type=jnp.float32)

    outs = pl.pallas_call(
        body,
        grid=(NB_TC,),
        in_specs=(
            [pl.BlockSpec((2, 128, DC), lambda i: (0, i, 0))] * (ngcn + 1)
            + [pl.BlockSpec((128, DC), lambda i: (i, 0))]
            + [pl.BlockSpec(w.shape, lambda i: (0, 0))
               for w in (wi2p, b2i, bg2, gcat)]
        ),
        out_specs=[
            pl.BlockSpec((128, 256), lambda i: (i, 0)),
            pl.BlockSpec((128, 32), lambda i: (i, 0)),
        ],
        out_shape=[
            jax.ShapeDtypeStruct((NPAD, 256), jnp.float32),
            jax.ShapeDtypeStruct((NPAD, 32), jnp.float32),
        ],
    )(*us, ysc, wi2p, b2i, bg2, gcat)
    return outs


def _final_stage(x3a_rs, wbig, bcfull):
    npr = x3a_rs.shape[0]

    def body(x_ref, w_ref, b_ref, out_ref):
        o = jax.lax.dot_general(x_ref[...], w_ref[...],
                                (((1,), (0,)), ((), ())),
                                preferred_element_type=jnp.float32)
        out_ref[...] = jax.nn.sigmoid(o + b_ref[...])

    out = pl.pallas_call(
        body,
        grid=(npr // 128,),
        in_specs=[
            pl.BlockSpec((128, x3a_rs.shape[1]), lambda i: (i, 0)),
            pl.BlockSpec(wbig.shape, lambda i: (0, 0)),
            pl.BlockSpec((128, 1), lambda i: (i, 0)),
        ],
        out_specs=pl.BlockSpec((128, 1), lambda i: (i, 0)),
        out_shape=jax.ShapeDtypeStruct((npr, 1), jnp.float32),
    )(x3a_rs, wbig, bcfull)
    return out


NB_POOL = 1792


def kernel(x, edge_weight, params, edge_index):
    p = params
    row, col = edge_index[0], edge_index[1]
    n = x.shape[0]
    nb = n // 30
    w = edge_weight[:, 0]

    pad_e = EPAD - E_EDGES
    pad_idx = N_NODES + (jnp.arange(pad_e, dtype=jnp.int32) % 64)
    rowp = jnp.concatenate([row, pad_idx]).reshape(-1, 128)
    colp = jnp.concatenate([col, pad_idx]).reshape(-1, 128)
    wx = jnp.broadcast_to(
        jnp.concatenate([w, jnp.zeros((pad_e,), jnp.float32)])[:, None],
        (EPAD, DC))
    z16 = jnp.zeros((NPAD, DC), jnp.float32)

    def pad_p(m):
        return jnp.zeros((NPAD, DC), jnp.float32).at[:n, :m.shape[1]].set(m)

    def padw16(m):
        return jnp.zeros((16, m.shape[1]), jnp.float32).at[:10].set(m)

    def padc16(m):
        return jnp.zeros((m.shape[0], 16), jnp.float32).at[:, :10].set(m)

    def padb16(v):
        return jnp.zeros((16,), jnp.float32).at[:10].set(v)

    mu = x.mean(axis=0)
    mx = x.max(axis=0)
    x0 = (x - mu) / (mx - mu)
    x0p = jnp.zeros((NPAD, 3), jnp.float32).at[:n].set(x0)

    chunks = _first_stage(x0p, p["gcn1_W2"], padw16(p["gin1_W1"]))

    def sc_and_selfterm(chunks, gin):
        ngcn = len(chunks) - 1
        wgts = [True] * ngcn + [False]
        u_all = _layer_pass(chunks, wgts, rowp, colp, wx, z16)
        us = [u_all[k] for k in range(len(chunks))]
        ysc = ((1.0 + p[gin + "_eps"][0]) * chunks[ngcn]
               + padb16(p[gin + "_b1"])[None, :])
        return us, ysc

    for l, nxt in ((1, 2), (2, 3)):
        gcn, gin = f"gcn{l}", f"gin{l}"
        us, ysc = sc_and_selfterm(chunks, gin)
        chunks = _combine_transform(
            us, ysc, padc16(p[gin + "_W2"]), p[gin + "_b2"][None, :],
            p[gcn + "_b2"][None, :], p[f"gcn{nxt}_W2"],
            padw16(p[f"gin{nxt}_W1"]), n)

    us, ysc = sc_and_selfterm(chunks, "gin3")
    gat_w = p["gat_W"][0]
    a0 = p["gat_att"][0, 0, 0]
    gcat = jnp.concatenate(
        [jnp.broadcast_to(gat_w[:, None], (256, 16)),
         jnp.broadcast_to((a0 * gat_w)[:, None], (256, 16))], axis=1)
    x3p, hl = _combine3(us, ysc, padc16(p["gin3_W2"]),
                        p["gin3_b2"][None, :], p["gcn3_b2"][None, :], gcat, n)

    hv = hl[:n, 0]
    lv = hl[:n, 16]
    l_ = jnp.where(lv >= 0, lv, 0.2 * lv)
    m = jnp.max(l_)
    ev = jnp.exp(l_ - m)
    q = hv * ev
    qe = pad_p(jnp.stack([q, ev], axis=1))
    u = _layer_pass([qe], [False], rowp, colp, wx, z16)[0]
    u = u[0] + u[1]
    z_norm = jnp.sum(u[:, 1])
    attw = u[:n, 0] / z_norm

    x3a = x3p[:n] * attw[:, None]
    wc = (p["lin2_W"] @ p["lin_W"])[0]
    bc = p["lin_b"] @ p["lin2_W"][0] + p["lin2_b"][0]
    xr = jnp.zeros((NB_POOL, 30 * 256), jnp.float32).at[:nb].set(
        x3a.reshape(nb, 30 * 256))
    wbig = jnp.tile(wc / 30.0, 30)[:, None]
    bcfull = jnp.full((NB_POOL, 1), bc, jnp.float32)
    out = _final_stage(xr, wbig, bcfull)
    return out[:nb, 0]


# speedup vs baseline: 6.4787x; 1.1718x over previous
import functools

import jax
import jax.numpy as jnp
from jax import lax
from jax.experimental import pallas as pl
from jax.experimental.pallas import tpu as pltpu
from jax.experimental.pallas import tpu_sc as plsc

N_NODES = 49980
NPAD = 50048
E_EDGES = 799680
NW = 32
PER_W = 26624
EPAD = NW * PER_W
B_WIN = 1024
BI = B_WIN // 128
NWIN = PER_W // B_WIN


DC = 16


def _layer_pass(p_chunks, weighted, row2, col2, wx, zeros):
    nch = len(p_chunks)
    mesh = plsc.VectorSubcoreMesh(core_axis_name="c", subcore_axis_name="s")

    def body(*refs):
        ps = refs[:nch]
        row_hbm, col_hbm, w_hbm, z_hbm, out_hbm = refs[nch:nch + 5]
        (ridx0, ridx1, cidx0, cidx1, wv0, wv1, rows0, rows1, acc,
         semg0, semg1, sems0, sems1) = refs[nch + 5:]
        cid = lax.axis_index("c")
        sid = lax.axis_index("s")
        base128 = (sid * 2 + cid) * (PER_W // 128)
        ridx = (ridx0, ridx1)
        cidx = (cidx0, cidx1)
        wv = (wv0, wv1)
        rows = (rows0, rows1)
        semg = (semg0, semg1)
        sems = (sems0, sems1)

        def drain_scatter(b):
            pltpu.make_async_copy(ps[0].at[pl.ds(0, B_WIN)],
                                  rows[b], sems[b]).wait()

        def prefetch(p_hbm, wgt, i, b):
            wrow = base128 + i * BI
            pltpu.sync_copy(row_hbm.at[pl.ds(wrow, BI)], ridx[b])
            for j in range(BI):
                pltpu.async_copy(p_hbm.at[ridx[b].at[j]],
                                 rows[b].at[pl.ds(j * 128, 128)], semg[b])
            if wgt:
                pltpu.sync_copy(w_hbm.at[pl.ds(wrow * 128, B_WIN)], wv[b])

        def compute(p_hbm, wgt, i, b):
            wrow = base128 + i * BI
            pltpu.make_async_copy(p_hbm.at[pl.ds(0, B_WIN)],
                                  rows[b], semg[b]).wait()
            if wgt:
                def mul8(g, c2):
                    for u in range(8):
                        r = g * 8 + u
                        rows[b][r, :] = rows[b][r, :] * wv[b][r, :]
                    return c2
                lax.fori_loop(0, B_WIN // 8, mul8, 0)
            pltpu.sync_copy(col_hbm.at[pl.ds(wrow, BI)], cidx[b])
            for j in range(BI):
                pltpu.sync_copy(rows[b].at[pl.ds(j * 128, 128)],
                                acc.at[cidx[b].at[j]], add=True)

        half = NWIN // 2
        for k in range(nch):
            p_hbm = ps[k]
            wgt = weighted[k]
            prefetch(p_hbm, wgt, 0, 0)
            plsc.subcore_barrier()

            @pl.when(sid == 0)
            def _zero():
                pltpu.sync_copy(z_hbm, acc)

            plsc.subcore_barrier()

            def pair(t, carry):
                prefetch(p_hbm, wgt, 2 * t + 1, 1)
                compute(p_hbm, wgt, 2 * t, 0)

                @pl.when(t + 1 < half)
                def _pf():
                    prefetch(p_hbm, wgt, 2 * t + 2, 0)

                compute(p_hbm, wgt, 2 * t + 1, 1)
                return carry

            lax.fori_loop(0, half, pair, 0)
            plsc.subcore_barrier()

            @pl.when(sid == 0)
            def _out():
                pltpu.sync_copy(acc, out_hbm.at[k].at[cid])

    k = pl.kernel(
        body,
        mesh=mesh,
        compiler_params=pltpu.CompilerParams(use_tc_tiling_on_sc=False),
        out_type=jax.ShapeDtypeStruct((nch, 2, NPAD, DC), jnp.float32),
        scratch_types=[
            pltpu.VMEM((BI, 128), jnp.int32),
            pltpu.VMEM((BI, 128), jnp.int32),
            pltpu.VMEM((BI, 128), jnp.int32),
            pltpu.VMEM((BI, 128), jnp.int32),
            pltpu.VMEM((B_WIN, DC), jnp.float32),
            pltpu.VMEM((B_WIN, DC), jnp.float32),
            pltpu.VMEM((B_WIN, DC), jnp.float32),
            pltpu.VMEM((B_WIN, DC), jnp.float32),
            pltpu.VMEM_SHARED((NPAD, DC), jnp.float32),
            pltpu.SemaphoreType.DMA,
            pltpu.SemaphoreType.DMA,
            pltpu.SemaphoreType.DMA,
            pltpu.SemaphoreType.DMA,
        ],
    )
    return k(*p_chunks, row2, col2, wx, zeros)


def _dot_t(a, b):
    return jax.lax.dot_general(a, b, (((1,), (1,)), ((), ())),
                               preferred_element_type=jnp.float32)


NB_TC = NPAD // 128


def _first_stage(x0p, wg, wy):
    dout = wg.shape[0]
    nch = dout // DC

    def body(x_ref, wg_ref, wy_ref, *outs):
        x = x_ref[...]
        zn = _dot_t(x, wg_ref[...])
        yn = _dot_t(x, wy_ref[...])
        for c in range(nch):
            outs[c][...] = zn[:, c * DC:(c + 1) * DC]
        outs[nch][...] = yn

    outs = pl.pallas_call(
        body,
        grid=(NB_TC,),
        in_specs=[
            pl.BlockSpec((128, x0p.shape[1]), lambda i: (i, 0)),
            pl.BlockSpec(wg.shape, lambda i: (0, 0)),
            pl.BlockSpec(wy.shape, lambda i: (0, 0)),
        ],
        out_specs=[pl.BlockSpec((128, DC), lambda i: (i, 0))] * (nch + 1),
        out_shape=[jax.ShapeDtypeStruct((NPAD, DC), jnp.float32)] * (nch + 1),
    )(x0p, wg, wy)
    return list(outs)


def _combine_transform(us, ysc, wi2p, b2i, bg2, wng, wny, n_valid):
    ngcn = len(us) - 1
    doutn = wng.shape[0]
    nchn = doutn // DC

    def body(*refs):
        u_refs = refs[:ngcn + 1]
        ysc_ref, wi2_ref, b2i_ref, bg2_ref, wg_ref, wy_ref = \
            refs[ngcn + 1:ngcn + 7]
        outs = refs[ngcn + 7:]
        pid = pl.program_id(0)
        g = jnp.concatenate([u[0] + u[1] for u in u_refs[:ngcn]], axis=1)
        g = g + bg2_ref[...]
        a16 = u_refs[ngcn][0] + u_refs[ngcn][1]
        t16 = jax.nn.relu(ysc_ref[...] + a16)
        i_ = jax.nn.relu(_dot_t(t16, wi2_ref[...]) + b2i_ref[...])
        xl = jnp.concatenate([g, i_], axis=1)
        xl = jnp.where(xl >= 0, xl, 0.2 * xl)
        rid = jax.lax.broadcasted_iota(jnp.int32, xl.shape, 0) + pid * 128
        xl = jnp.where(rid < n_valid, xl, 0.0)
        zn = _dot_t(xl, wg_ref[...])
        yn = _dot_t(xl, wy_ref[...])
        for c in range(nchn):
            outs[c][...] = zn[:, c * DC:(c + 1) * DC]
        outs[nchn][...] = yn

    outs = pl.pallas_call(
        body,
        grid=(NB_TC,),
        in_specs=(
            [pl.BlockSpec((2, 128, DC), lambda i: (0, i, 0))] * (ngcn + 1)
            + [pl.BlockSpec((128, DC), lambda i: (i, 0))]
            + [pl.BlockSpec(w.shape, lambda i: (0, 0))
               for w in (wi2p, b2i, bg2, wng, wny)]
        ),
        out_specs=[pl.BlockSpec((128, DC), lambda i: (i, 0))] * (nchn + 1),
        out_shape=[jax.ShapeDtypeStruct((NPAD, DC), jnp.float32)]
        * (nchn + 1),
    )(*us, ysc, wi2p, b2i, bg2, wng, wny)
    return list(outs)


def _combine3(us, ysc, wi2p, b2i, bg2, gcat, n_valid):
    ngcn = len(us) - 1

    def body(*refs):
        u_refs = refs[:ngcn + 1]
        ysc_ref, wi2_ref, b2i_ref, bg2_ref, g_ref = refs[ngcn + 1:ngcn + 6]
        x3_ref, hl_ref = refs[ngcn + 6:]
        pid = pl.program_id(0)
        g = jnp.concatenate([u[0] + u[1] for u in u_refs[:ngcn]], axis=1)
        g = g + bg2_ref[...]
        a16 = u_refs[ngcn][0] + u_refs[ngcn][1]
        t16 = jax.nn.relu(ysc_ref[...] + a16)
        i_ = jax.nn.relu(_dot_t(t16, wi2_ref[...]) + b2i_ref[...])
        xl = jnp.concatenate([g, i_], axis=1)
        xl = jnp.where(xl >= 0, xl, 0.2 * xl)
        rid = jax.lax.broadcasted_iota(jnp.int32, xl.shape, 0) + pid * 128
        xl = jnp.where(rid < n_valid, xl, 0.0)
        x3_ref[...] = xl
        hl_ref[...] = jax.lax.dot_general(
            xl, g_ref[...], (((1,), (0,)), ((), ())),
            preferred_element_type=jnp.float32)

    outs = pl.pallas_call(
        body,
        grid=(NB_TC,),
        in_specs=(
            [pl.BlockSpec((2, 128, DC), lambda i: (0, i, 0))] * (ngcn + 1)
            + [pl.BlockSpec((128, DC), lambda i: (i, 0))]
            + [pl.BlockSpec(w.shape, lambda i: (0, 0))
               for w in (wi2p, b2i, bg2, gcat)]
        ),
        out_specs=[
            pl.BlockSpec((128, 256), lambda i: (i, 0)),
            pl.BlockSpec((128, 32), lambda i: (i, 0)),
        ],
        out_shape=[
            jax.ShapeDtypeStruct((NPAD, 256), jnp.float32),
            jax.ShapeDtypeStruct((NPAD, 32), jnp.float32),
        ],
    )(*us, ysc, wi2p, b2i, bg2, gcat)
    return outs


def _final_stage(x3a_rs, wbig, bcfull):
    npr = x3a_rs.shape[0]

    def body(x_ref, w_ref, b_ref, out_ref):
        o = jax.lax.dot_general(x_ref[...], w_ref[...],
                                (((1,), (0,)), ((), ())),
                                preferred_element_type=jnp.float32)
        out_ref[...] = jax.nn.sigmoid(o + b_ref[...])

    out = pl.pallas_call(
        body,
        grid=(npr // 128,),
        in_specs=[
            pl.BlockSpec((128, x3a_rs.shape[1]), lambda i: (i, 0)),
            pl.BlockSpec(wbig.shape, lambda i: (0, 0)),
            pl.BlockSpec((128, 1), lambda i: (i, 0)),
        ],
        out_specs=pl.BlockSpec((128, 1), lambda i: (i, 0)),
        out_shape=jax.ShapeDtypeStruct((npr, 1), jnp.float32),
    )(x3a_rs, wbig, bcfull)
    return out


NB_POOL = 1792


def kernel(x, edge_weight, params, edge_index):
    p = params
    row, col = edge_index[0], edge_index[1]
    n = x.shape[0]
    nb = n // 30
    w = edge_weight[:, 0]

    pad_e = EPAD - E_EDGES
    pad_idx = N_NODES + (jnp.arange(pad_e, dtype=jnp.int32) % 64)
    rowp = jnp.concatenate([row, pad_idx]).reshape(-1, 128)
    colp = jnp.concatenate([col, pad_idx]).reshape(-1, 128)
    wx = jnp.broadcast_to(
        jnp.concatenate([w, jnp.zeros((pad_e,), jnp.float32)])[:, None],
        (EPAD, DC))
    z16 = jnp.zeros((NPAD, DC), jnp.float32)

    def pad_p(m):
        return jnp.zeros((NPAD, DC), jnp.float32).at[:n, :m.shape[1]].set(m)

    def padw16(m):
        return jnp.zeros((16, m.shape[1]), jnp.float32).at[:10].set(m)

    def padc16(m):
        return jnp.zeros((m.shape[0], 16), jnp.float32).at[:, :10].set(m)

    def padb16(v):
        return jnp.zeros((16,), jnp.float32).at[:10].set(v)

    mu = x.mean(axis=0)
    mx = x.max(axis=0)
    x0 = (x - mu) / (mx - mu)
    x0p = jnp.zeros((NPAD, 3), jnp.float32).at[:n].set(x0)

    chunks = _first_stage(x0p, p["gcn1_W2"], padw16(p["gin1_W1"]))

    def sc_and_selfterm(chunks, gin):
        ngcn = len(chunks) - 1
        wgts = [True] * ngcn + [False]
        us = [_layer_pass([c], [wg], rowp, colp, wx, z16)[0]
              for c, wg in zip(chunks, wgts)]
        ysc = ((1.0 + p[gin + "_eps"][0]) * chunks[ngcn]
               + padb16(p[gin + "_b1"])[None, :])
        return us, ysc

    for l, nxt in ((1, 2), (2, 3)):
        gcn, gin = f"gcn{l}", f"gin{l}"
        us, ysc = sc_and_selfterm(chunks, gin)
        chunks = _combine_transform(
            us, ysc, padc16(p[gin + "_W2"]), p[gin + "_b2"][None, :],
            p[gcn + "_b2"][None, :], p[f"gcn{nxt}_W2"],
            padw16(p[f"gin{nxt}_W1"]), n)

    us, ysc = sc_and_selfterm(chunks, "gin3")
    gat_w = p["gat_W"][0]
    a0 = p["gat_att"][0, 0, 0]
    gcat = jnp.concatenate(
        [jnp.broadcast_to(gat_w[:, None], (256, 16)),
         jnp.broadcast_to((a0 * gat_w)[:, None], (256, 16))], axis=1)
    x3p, hl = _combine3(us, ysc, padc16(p["gin3_W2"]),
                        p["gin3_b2"][None, :], p["gcn3_b2"][None, :], gcat, n)

    hv = hl[:n, 0]
    lv = hl[:n, 16]
    l_ = jnp.where(lv >= 0, lv, 0.2 * lv)
    m = jnp.max(l_)
    ev = jnp.exp(l_ - m)
    q = hv * ev
    qe = pad_p(jnp.stack([q, ev], axis=1))
    u = _layer_pass([qe], [False], rowp, colp, wx, z16)[0]
    u = u[0] + u[1]
    z_norm = jnp.sum(u[:, 1])
    attw = u[:n, 0] / z_norm

    x3a = x3p[:n] * attw[:, None]
    wc = (p["lin2_W"] @ p["lin_W"])[0]
    bc = p["lin_b"] @ p["lin2_W"][0] + p["lin2_b"][0]
    xr = jnp.zeros((NB_POOL, 30 * 256), jnp.float32).at[:nb].set(
        x3a.reshape(nb, 30 * 256))
    wbig = jnp.tile(wc / 30.0, 30)[:, None]
    bcfull = jnp.full((NB_POOL, 1), bc, jnp.float32)
    out = _final_stage(xr, wbig, bcfull)
    return out[:nb, 0]


# speedup vs baseline: 6.7724x; 1.0453x over previous
import functools

import jax
import jax.numpy as jnp
from jax import lax
from jax.experimental import pallas as pl
from jax.experimental.pallas import tpu as pltpu
from jax.experimental.pallas import tpu_sc as plsc

N_NODES = 49980
NPAD = 50048
E_EDGES = 799680
NW = 32
PER_W = 25600
EPAD = NW * PER_W
B_WIN = 1024
BI = B_WIN // 128
NWIN = PER_W // B_WIN


DC = 16


def _layer_pass(p_chunks, weighted, row2, col2, wx, zeros):
    nch = len(p_chunks)
    mesh = plsc.VectorSubcoreMesh(core_axis_name="c", subcore_axis_name="s")

    def body(*refs):
        ps = refs[:nch]
        row_hbm, col_hbm, w_hbm, z_hbm, out_hbm = refs[nch:nch + 5]
        (ridx0, ridx1, cidx0, cidx1, wv0, wv1, rows0, rows1, acc,
         semg0, semg1, sems0, sems1) = refs[nch + 5:]
        cid = lax.axis_index("c")
        sid = lax.axis_index("s")
        base128 = (sid * 2 + cid) * (PER_W // 128)
        ridx = (ridx0, ridx1)
        cidx = (cidx0, cidx1)
        wv = (wv0, wv1)
        rows = (rows0, rows1)
        semg = (semg0, semg1)
        sems = (sems0, sems1)

        def drain_scatter(b):
            pltpu.make_async_copy(ps[0].at[pl.ds(0, B_WIN)],
                                  rows[b], sems[b]).wait()

        def prefetch(p_hbm, wgt, i, b):
            wrow = base128 + i * BI
            pltpu.sync_copy(row_hbm.at[pl.ds(wrow, BI)], ridx[b])
            for j in range(BI):
                pltpu.async_copy(p_hbm.at[ridx[b].at[j]],
                                 rows[b].at[pl.ds(j * 128, 128)], semg[b])
            if wgt:
                pltpu.sync_copy(w_hbm.at[pl.ds(wrow * 128, B_WIN)], wv[b])

        def compute(p_hbm, wgt, i, b):
            wrow = base128 + i * BI
            pltpu.make_async_copy(p_hbm.at[pl.ds(0, B_WIN)],
                                  rows[b], semg[b]).wait()
            if wgt:
                def mul8(g, c2):
                    for u in range(8):
                        r = g * 8 + u
                        rows[b][r, :] = rows[b][r, :] * wv[b][r, :]
                    return c2
                lax.fori_loop(0, B_WIN // 8, mul8, 0)
            pltpu.sync_copy(col_hbm.at[pl.ds(wrow, BI)], cidx[b])
            for j in range(BI):
                pltpu.sync_copy(rows[b].at[pl.ds(j * 128, 128)],
                                acc.at[cidx[b].at[j]], add=True)

        half = NWIN // 2
        for k in range(nch):
            p_hbm = ps[k]
            wgt = weighted[k]
            prefetch(p_hbm, wgt, 0, 0)
            plsc.subcore_barrier()

            @pl.when(sid == 0)
            def _zero():
                pltpu.sync_copy(z_hbm, acc)

            plsc.subcore_barrier()

            def pair(t, carry):
                prefetch(p_hbm, wgt, 2 * t + 1, 1)
                compute(p_hbm, wgt, 2 * t, 0)

                @pl.when(2 * t + 2 < NWIN)
                def _pf():
                    prefetch(p_hbm, wgt, 2 * t + 2, 0)

                compute(p_hbm, wgt, 2 * t + 1, 1)
                return carry

            lax.fori_loop(0, half, pair, 0)
            if NWIN % 2:
                compute(p_hbm, wgt, NWIN - 1, 0)
            plsc.subcore_barrier()

            @pl.when(sid == 0)
            def _out():
                pltpu.sync_copy(acc, out_hbm.at[k].at[cid])

    k = pl.kernel(
        body,
        mesh=mesh,
        compiler_params=pltpu.CompilerParams(use_tc_tiling_on_sc=False),
        out_type=jax.ShapeDtypeStruct((nch, 2, NPAD, DC), jnp.float32),
        scratch_types=[
            pltpu.VMEM((BI, 128), jnp.int32),
            pltpu.VMEM((BI, 128), jnp.int32),
            pltpu.VMEM((BI, 128), jnp.int32),
            pltpu.VMEM((BI, 128), jnp.int32),
            pltpu.VMEM((B_WIN, DC), jnp.float32),
            pltpu.VMEM((B_WIN, DC), jnp.float32),
            pltpu.VMEM((B_WIN, DC), jnp.float32),
            pltpu.VMEM((B_WIN, DC), jnp.float32),
            pltpu.VMEM_SHARED((NPAD, DC), jnp.float32),
            pltpu.SemaphoreType.DMA,
            pltpu.SemaphoreType.DMA,
            pltpu.SemaphoreType.DMA,
            pltpu.SemaphoreType.DMA,
        ],
    )
    return k(*p_chunks, row2, col2, wx, zeros)


def _dot_t(a, b):
    return jax.lax.dot_general(a, b, (((1,), (1,)), ((), ())),
                               preferred_element_type=jnp.float32)


NB_TC = NPAD // 128


def _first_stage(x0p, wg, wy):
    dout = wg.shape[0]
    nch = dout // DC

    def body(x_ref, wg_ref, wy_ref, *outs):
        x = x_ref[...]
        zn = _dot_t(x, wg_ref[...])
        yn = _dot_t(x, wy_ref[...])
        for c in range(nch):
            outs[c][...] = zn[:, c * DC:(c + 1) * DC]
        outs[nch][...] = yn

    outs = pl.pallas_call(
        body,
        grid=(NB_TC,),
        in_specs=[
            pl.BlockSpec((128, x0p.shape[1]), lambda i: (i, 0)),
            pl.BlockSpec(wg.shape, lambda i: (0, 0)),
            pl.BlockSpec(wy.shape, lambda i: (0, 0)),
        ],
        out_specs=[pl.BlockSpec((128, DC), lambda i: (i, 0))] * (nch + 1),
        out_shape=[jax.ShapeDtypeStruct((NPAD, DC), jnp.float32)] * (nch + 1),
    )(x0p, wg, wy)
    return list(outs)


def _combine_transform(us, ysc, wi2p, b2i, bg2, wng, wny, n_valid):
    ngcn = len(us) - 1
    doutn = wng.shape[0]
    nchn = doutn // DC

    def body(*refs):
        u_refs = refs[:ngcn + 1]
        ysc_ref, wi2_ref, b2i_ref, bg2_ref, wg_ref, wy_ref = \
            refs[ngcn + 1:ngcn + 7]
        outs = refs[ngcn + 7:]
        pid = pl.program_id(0)
        g = jnp.concatenate([u[0] + u[1] for u in u_refs[:ngcn]], axis=1)
        g = g + bg2_ref[...]
        a16 = u_refs[ngcn][0] + u_refs[ngcn][1]
        t16 = jax.nn.relu(ysc_ref[...] + a16)
        i_ = jax.nn.relu(_dot_t(t16, wi2_ref[...]) + b2i_ref[...])
        xl = jnp.concatenate([g, i_], axis=1)
        xl = jnp.where(xl >= 0, xl, 0.2 * xl)
        rid = jax.lax.broadcasted_iota(jnp.int32, xl.shape, 0) + pid * 128
        xl = jnp.where(rid < n_valid, xl, 0.0)
        zn = _dot_t(xl, wg_ref[...])
        yn = _dot_t(xl, wy_ref[...])
        for c in range(nchn):
            outs[c][...] = zn[:, c * DC:(c + 1) * DC]
        outs[nchn][...] = yn

    outs = pl.pallas_call(
        body,
        grid=(NB_TC,),
        in_specs=(
            [pl.BlockSpec((2, 128, DC), lambda i: (0, i, 0))] * (ngcn + 1)
            + [pl.BlockSpec((128, DC), lambda i: (i, 0))]
            + [pl.BlockSpec(w.shape, lambda i: (0, 0))
               for w in (wi2p, b2i, bg2, wng, wny)]
        ),
        out_specs=[pl.BlockSpec((128, DC), lambda i: (i, 0))] * (nchn + 1),
        out_shape=[jax.ShapeDtypeStruct((NPAD, DC), jnp.float32)]
        * (nchn + 1),
    )(*us, ysc, wi2p, b2i, bg2, wng, wny)
    return list(outs)


def _combine3(us, ysc, wi2p, b2i, bg2, gcat, n_valid):
    ngcn = len(us) - 1

    def body(*refs):
        u_refs = refs[:ngcn + 1]
        ysc_ref, wi2_ref, b2i_ref, bg2_ref, g_ref = refs[ngcn + 1:ngcn + 6]
        x3_ref, hl_ref = refs[ngcn + 6:]
        pid = pl.program_id(0)
        g = jnp.concatenate([u[0] + u[1] for u in u_refs[:ngcn]], axis=1)
        g = g + bg2_ref[...]
        a16 = u_refs[ngcn][0] + u_refs[ngcn][1]
        t16 = jax.nn.relu(ysc_ref[...] + a16)
        i_ = jax.nn.relu(_dot_t(t16, wi2_ref[...]) + b2i_ref[...])
        xl = jnp.concatenate([g, i_], axis=1)
        xl = jnp.where(xl >= 0, xl, 0.2 * xl)
        rid = jax.lax.broadcasted_iota(jnp.int32, xl.shape, 0) + pid * 128
        xl = jnp.where(rid < n_valid, xl, 0.0)
        x3_ref[...] = xl
        hl_ref[...] = jax.lax.dot_general(
            xl, g_ref[...], (((1,), (0,)), ((), ())),
            preferred_element_type=jnp.float32)

    outs = pl.pallas_call(
        body,
        grid=(NB_TC,),
        in_specs=(
            [pl.BlockSpec((2, 128, DC), lambda i: (0, i, 0))] * (ngcn + 1)
            + [pl.BlockSpec((128, DC), lambda i: (i, 0))]
            + [pl.BlockSpec(w.shape, lambda i: (0, 0))
               for w in (wi2p, b2i, bg2, gcat)]
        ),
        out_specs=[
            pl.BlockSpec((128, 256), lambda i: (i, 0)),
            pl.BlockSpec((128, 32), lambda i: (i, 0)),
        ],
        out_shape=[
            jax.ShapeDtypeStruct((NPAD, 256), jnp.float32),
            jax.ShapeDtypeStruct((NPAD, 32), jnp.float32),
        ],
    )(*us, ysc, wi2p, b2i, bg2, gcat)
    return outs


def _final_stage(x3a_rs, wbig, bcfull):
    npr = x3a_rs.shape[0]

    def body(x_ref, w_ref, b_ref, out_ref):
        o = jax.lax.dot_general(x_ref[...], w_ref[...],
                                (((1,), (0,)), ((), ())),
                                preferred_element_type=jnp.float32)
        out_ref[...] = jax.nn.sigmoid(o + b_ref[...])

    out = pl.pallas_call(
        body,
        grid=(npr // 128,),
        in_specs=[
            pl.BlockSpec((128, x3a_rs.shape[1]), lambda i: (i, 0)),
            pl.BlockSpec(wbig.shape, lambda i: (0, 0)),
            pl.BlockSpec((128, 1), lambda i: (i, 0)),
        ],
        out_specs=pl.BlockSpec((128, 1), lambda i: (i, 0)),
        out_shape=jax.ShapeDtypeStruct((npr, 1), jnp.float32),
    )(x3a_rs, wbig, bcfull)
    return out


NB_POOL = 1792


def kernel(x, edge_weight, params, edge_index):
    p = params
    row, col = edge_index[0], edge_index[1]
    n = x.shape[0]
    nb = n // 30
    w = edge_weight[:, 0]

    pad_e = EPAD - E_EDGES
    pad_idx = N_NODES + (jnp.arange(pad_e, dtype=jnp.int32) % 64)
    rowp = jnp.concatenate([row, pad_idx]).reshape(-1, 128)
    colp = jnp.concatenate([col, pad_idx]).reshape(-1, 128)
    wx = jnp.broadcast_to(
        jnp.concatenate([w, jnp.zeros((pad_e,), jnp.float32)])[:, None],
        (EPAD, DC))
    z16 = jnp.zeros((NPAD, DC), jnp.float32)

    def pad_p(m):
        return jnp.zeros((NPAD, DC), jnp.float32).at[:n, :m.shape[1]].set(m)

    def padw16(m):
        return jnp.zeros((16, m.shape[1]), jnp.float32).at[:10].set(m)

    def padc16(m):
        return jnp.zeros((m.shape[0], 16), jnp.float32).at[:, :10].set(m)

    def padb16(v):
        return jnp.zeros((16,), jnp.float32).at[:10].set(v)

    mu = x.mean(axis=0)
    mx = x.max(axis=0)
    x0 = (x - mu) / (mx - mu)
    x0p = jnp.zeros((NPAD, 3), jnp.float32).at[:n].set(x0)

    chunks = _first_stage(x0p, p["gcn1_W2"], padw16(p["gin1_W1"]))

    def sc_and_selfterm(chunks, gin):
        ngcn = len(chunks) - 1
        wgts = [True] * ngcn + [False]
        us = [_layer_pass([c], [wg], rowp, colp, wx, z16)[0]
              for c, wg in zip(chunks, wgts)]
        ysc = ((1.0 + p[gin + "_eps"][0]) * chunks[ngcn]
               + padb16(p[gin + "_b1"])[None, :])
        return us, ysc

    for l, nxt in ((1, 2), (2, 3)):
        gcn, gin = f"gcn{l}", f"gin{l}"
        us, ysc = sc_and_selfterm(chunks, gin)
        chunks = _combine_transform(
            us, ysc, padc16(p[gin + "_W2"]), p[gin + "_b2"][None, :],
            p[gcn + "_b2"][None, :], p[f"gcn{nxt}_W2"],
            padw16(p[f"gin{nxt}_W1"]), n)

    us, ysc = sc_and_selfterm(chunks, "gin3")
    gat_w = p["gat_W"][0]
    a0 = p["gat_att"][0, 0, 0]
    gcat = jnp.concatenate(
        [jnp.broadcast_to(gat_w[:, None], (256, 16)),
         jnp.broadcast_to((a0 * gat_w)[:, None], (256, 16))], axis=1)
    x3p, hl = _combine3(us, ysc, padc16(p["gin3_W2"]),
                        p["gin3_b2"][None, :], p["gcn3_b2"][None, :], gcat, n)

    hv = hl[:n, 0]
    lv = hl[:n, 16]
    l_ = jnp.where(lv >= 0, lv, 0.2 * lv)
    m = jnp.max(l_)
    ev = jnp.exp(l_ - m)
    q = hv * ev
    qe = pad_p(jnp.stack([q, ev], axis=1))
    u = _layer_pass([qe], [False], rowp, colp, wx, z16)[0]
    u = u[0] + u[1]
    z_norm = jnp.sum(u[:, 1])
    attw = u[:n, 0] / z_norm

    x3a = x3p[:n] * attw[:, None]
    wc = (p["lin2_W"] @ p["lin_W"])[0]
    bc = p["lin_b"] @ p["lin2_W"][0] + p["lin2_b"][0]
    xr = jnp.zeros((NB_POOL, 30 * 256), jnp.float32).at[:nb].set(
        x3a.reshape(nb, 30 * 256))
    wbig = jnp.tile(wc / 30.0, 30)[:, None]
    bcfull = jnp.full((NB_POOL, 1), bc, jnp.float32)
    out = _final_stage(xr, wbig, bcfull)
    return out[:nb, 0]
